# baseline (speedup 1.0000x reference)
import functools
import jax
import jax.numpy as jnp
from jax import lax
from jax.experimental import pallas as pl
from jax.experimental.pallas import tpu as pltpu
from jax.experimental.pallas import tpu_sc as plsc

D = 64
NC = 2
NS = 16
NW = NC * NS
CHUNK = 128
NBUF = 4


def _embedding_body(x_hbm, w_hbm, out_hbm, idx_v, rows_v, sems):
    wid = lax.axis_index("s") * NC + lax.axis_index("c")
    chunks_per_w = idx_v.shape[0]
    base = wid * chunks_per_w
    pltpu.sync_copy(x_hbm.at[pl.ds(base, chunks_per_w)], idx_v)

    def gather(slot, chunk):
        return pltpu.make_async_copy(
            w_hbm.at[idx_v.at[chunk]], rows_v.at[slot], sems.at[slot])

    for b in range(NBUF):
        gather(b, b).start()

    def body(k, _):
        g = k * NBUF
        for b in range(NBUF):
            i = g + b
            gather(b, i).wait()
            pltpu.sync_copy(rows_v.at[b], out_hbm.at[base + i])
            gather(b, i + NBUF).start()
        return _

    lax.fori_loop(0, (chunks_per_w - NBUF) // NBUF, body, None)

    for b in range(NBUF):
        i = chunks_per_w - NBUF + b
        gather(b, i).wait()
        pltpu.sync_copy(rows_v.at[b], out_hbm.at[base + i])


def _make_call(n_chunks):
    chunks_per_w = n_chunks // NW
    mesh = plsc.VectorSubcoreMesh(core_axis_name="c", subcore_axis_name="s")
    return pl.kernel(
        _embedding_body,
        out_type=jax.ShapeDtypeStruct((n_chunks, CHUNK, D), jnp.float32),
        mesh=mesh,
        scratch_types=[
            pltpu.VMEM((chunks_per_w, CHUNK), jnp.int32),
            pltpu.VMEM((NBUF, CHUNK, D), jnp.float32),
            pltpu.SemaphoreType.DMA((NBUF,)),
        ],
        compiler_params=pltpu.CompilerParams(use_tc_tiling_on_sc=False),
    )


@jax.jit
def kernel(x, weight):
    s0, s1 = x.shape
    n = s0 * s1
    assert n % (NW * CHUNK) == 0
    xc = x.astype(jnp.int32).reshape(n // CHUNK, CHUNK)
    out = _make_call(n // CHUNK)(xc, weight)
    return out.reshape(s0, s1, D)


# speedup vs baseline: 1.0010x; 1.0010x over previous
import functools
import jax
import jax.numpy as jnp
from jax import lax
from jax.experimental import pallas as pl
from jax.experimental.pallas import tpu as pltpu
from jax.experimental.pallas import tpu_sc as plsc

D = 64
NC = 2
NS = 16
NW = NC * NS
CHUNK = 128
M = 8
H = M // 2


def _embedding_body(x_hbm, w_hbm, out_hbm, idx_v, rows_v, gsem, ssem):
    wid = lax.axis_index("s") * NC + lax.axis_index("c")
    n = idx_v.shape[0]
    base = wid * n
    pltpu.sync_copy(x_hbm.at[pl.ds(base, n)], idx_v)

    def gather(slot, chunk):
        return pltpu.make_async_copy(
            w_hbm.at[idx_v.at[chunk]], rows_v.at[slot], gsem.at[slot])

    def store(slot, chunk):
        return pltpu.make_async_copy(
            rows_v.at[slot], out_hbm.at[base + chunk], ssem.at[slot])

    for i in range(H):
        gather(i % M, i).start()
    for i in range(H):
        gather((i + H) % M, i + H).start()
        gather(i % M, i).wait()
        store(i % M, i).start()

    def body(k, _):
        first = H + k * M
        for t in range(M):
            i = first + t
            s_cur = (H + t) % M
            s_new = (H + t + H) % M
            store(s_new, i - H).wait()
            gather(s_new, i + H).start()
            gather(s_cur, i).wait()
            store(s_cur, i).start()
        return _

    lax.fori_loop(0, (n - M) // M, body, None)

    for t in range(H):
        i = n - H + t
        s_cur = (i % M)
        store((i + H) % M, i - H).wait()
        gather(s_cur, i).wait()
        store(s_cur, i).start()
    for t in range(H):
        i = n - H + t
        store(i % M, i).wait()


def _make_call(n_chunks):
    chunks_per_w = n_chunks // NW
    mesh = plsc.VectorSubcoreMesh(core_axis_name="c", subcore_axis_name="s")
    return pl.kernel(
        _embedding_body,
        out_type=jax.ShapeDtypeStruct((n_chunks, CHUNK, D), jnp.float32),
        mesh=mesh,
        scratch_types=[
            pltpu.VMEM((chunks_per_w, CHUNK), jnp.int32),
            pltpu.VMEM((M, CHUNK, D), jnp.float32),
            pltpu.SemaphoreType.DMA((M,)),
            pltpu.SemaphoreType.DMA((M,)),
        ],
        compiler_params=pltpu.CompilerParams(use_tc_tiling_on_sc=False),
    )


@jax.jit
def kernel(x, weight):
    s0, s1 = x.shape
    n = s0 * s1
    assert n % (NW * CHUNK) == 0
    xc = x.astype(jnp.int32).reshape(n // CHUNK, CHUNK)
    out = _make_call(n // CHUNK)(xc, weight)
    return out.reshape(s0, s1, D)
